# Initial kernel scaffold; baseline (speedup 1.0000x reference)
import jax
import jax.numpy as jnp
from jax.experimental import pallas as pl


def kernel(x, edge_index, edge_weight, mask_idx, y, W1, b1, W2, b2):
    raise NotImplementedError("write your pallas kernel here")


# speedup vs baseline: 12.2732x; 12.2732x over previous
import functools

import jax
import jax.numpy as jnp
from jax import lax
from jax.experimental import pallas as pl
from jax.experimental.pallas import tpu as pltpu
from jax.experimental.pallas import tpu_sc as plsc

N = 10000
E = 320000
D = 128
NMASK = 1000

NC = 2
NS = 16
NW = NC * NS
EPW = E // NW
CH = 128
NFULL = EPW // CH
TAIL = EPW - NFULL * CH
RPT = N // NS
ZCH = 125

f32 = jnp.float32
i32 = jnp.int32

_mesh = plsc.VectorSubcoreMesh(core_axis_name="c", subcore_axis_name="s")


@functools.partial(
    pl.kernel,
    out_type=jax.ShapeDtypeStruct((NC * N,), f32),
    mesh=_mesh,
    scratch_types=[
        pltpu.VMEM((CH,), i32),
        pltpu.VMEM((TAIL,), i32),
        pltpu.VMEM((CH,), f32),
        pltpu.VMEM((1024,), f32),
        pltpu.VMEM_SHARED((N,), f32),
    ],
)
def _deg_kernel(dst_hbm, ew_hbm, out_hbm, idx_d, idx_t, ewb, zb, sdeg):
    c = lax.axis_index("c")
    s = lax.axis_index("s")
    wid = s * NC + c

    def zb_body(i, carry):
        zb[pl.ds(i * 16, 16)] = jnp.zeros((16,), f32)
        return carry

    lax.fori_loop(0, 64, zb_body, 0)

    @pl.when(s < 10)
    def _():
        pltpu.sync_copy(zb.at[pl.ds(0, 1000)], sdeg.at[pl.ds(s * 1000, 1000)])

    plsc.subcore_barrier()

    base0 = wid * EPW

    def chunk(i, carry):
        base = base0 + i * CH
        pltpu.sync_copy(dst_hbm.at[pl.ds(base, CH)], idx_d)
        pltpu.sync_copy(ew_hbm.at[pl.ds(base, CH)], ewb)
        pltpu.sync_copy(ewb, sdeg.at[idx_d], add=True)
        return carry

    lax.fori_loop(0, NFULL, chunk, 0)

    baset = base0 + NFULL * CH
    pltpu.sync_copy(dst_hbm.at[pl.ds(baset, TAIL)], idx_t)
    pltpu.sync_copy(ew_hbm.at[pl.ds(baset, TAIL)], ewb.at[pl.ds(0, TAIL)])
    pltpu.sync_copy(ewb.at[pl.ds(0, TAIL)], sdeg.at[idx_t], add=True)

    plsc.subcore_barrier()

    @pl.when(s < 10)
    def _():
        pltpu.sync_copy(sdeg.at[pl.ds(s * 1000, 1000)], zb.at[pl.ds(0, 1000)])
        pltpu.sync_copy(zb.at[pl.ds(0, 1000)],
                        out_hbm.at[pl.ds(c * N + s * 1000, 1000)])


@functools.partial(
    pl.kernel,
    out_type=jax.ShapeDtypeStruct((NC, N, D), f32),
    mesh=_mesh,
    scratch_types=[
        pltpu.VMEM((CH,), i32),
        pltpu.VMEM((CH,), i32),
        pltpu.VMEM((TAIL,), i32),
        pltpu.VMEM((TAIL,), i32),
        pltpu.VMEM((CH,), f32),
        pltpu.VMEM((TAIL,), f32),
        pltpu.VMEM((CH, D), f32),
        pltpu.VMEM_SHARED((N, D), f32),
        pltpu.SemaphoreType.DMA,
    ],
)
def _agg_kernel(src_hbm, dst_hbm, ew_hbm, g_hbm, out_hbm,
                idx_s, idx_d, idx_st, idx_dt, ewb, ewt, rows, acc, sem):
    c = lax.axis_index("c")
    s = lax.axis_index("s")
    wid = s * NC + c

    def zrow(i, carry):
        for k8 in range(8):
            rows[i, pl.ds(k8 * 16, 16)] = jnp.zeros((16,), f32)
        return carry

    lax.fori_loop(0, CH, zrow, 0)

    rbase = s * RPT
    for k in range(5):
        pltpu.sync_copy(rows.at[pl.ds(0, ZCH)],
                        acc.at[pl.ds(rbase + k * ZCH, ZCH)])
    plsc.subcore_barrier()

    base0 = wid * EPW

    def scale_full(jv, carry):
        ew16 = ewb[pl.ds(jv * 16, 16)]
        for lane in range(16):
            sc = ew16[lane]
            j = jv * 16 + lane
            for k8 in range(8):
                sl = pl.ds(k8 * 16, 16)
                rows[j, sl] = rows[j, sl] * sc
        return carry

    def chunk(i, carry):
        base = base0 + i * CH
        pltpu.sync_copy(src_hbm.at[pl.ds(base, CH)], idx_s)
        pltpu.sync_copy(dst_hbm.at[pl.ds(base, CH)], idx_d)
        pltpu.sync_copy(ew_hbm.at[pl.ds(base, CH)], ewb)
        pltpu.async_copy(g_hbm.at[idx_s], rows, sem).wait()
        lax.fori_loop(0, CH // 16, scale_full, 0)
        pltpu.sync_copy(rows, acc.at[idx_d], add=True)
        return carry

    lax.fori_loop(0, NFULL, chunk, 0)

    baset = base0 + NFULL * CH
    pltpu.sync_copy(src_hbm.at[pl.ds(baset, TAIL)], idx_st)
    pltpu.sync_copy(dst_hbm.at[pl.ds(baset, TAIL)], idx_dt)
    pltpu.sync_copy(ew_hbm.at[pl.ds(baset, TAIL)], ewt)
    pltpu.async_copy(g_hbm.at[idx_st], rows.at[pl.ds(0, TAIL)], sem).wait()

    ew16t = ewt[...]
    for lane in range(16):
        sc = ew16t[lane]
        for k8 in range(8):
            sl = pl.ds(k8 * 16, 16)
            rows[lane, sl] = rows[lane, sl] * sc
    pltpu.sync_copy(rows.at[pl.ds(0, TAIL)], acc.at[idx_dt], add=True)

    plsc.subcore_barrier()
    obase = s * 624
    off = 0
    for sz in (128, 128, 128, 128, 112):
        pltpu.sync_copy(acc.at[pl.ds(obase + off, sz)], rows.at[pl.ds(0, sz)])
        pltpu.sync_copy(rows.at[pl.ds(0, sz)],
                        out_hbm.at[c, pl.ds(obase + off, sz)])
        off += sz

    @pl.when(s == 0)
    def _():
        pltpu.sync_copy(acc.at[pl.ds(9984, 16)], rows.at[pl.ds(0, 16)])
        pltpu.sync_copy(rows.at[pl.ds(0, 16)], out_hbm.at[c, pl.ds(9984, 16)])


MW = 25
MR = 40


@functools.partial(
    pl.kernel,
    out_type=(jax.ShapeDtypeStruct((NMASK, D), f32),
              jax.ShapeDtypeStruct((NMASK,), i32)),
    mesh=_mesh,
    scratch_types=[
        pltpu.VMEM((MR,), i32),
        pltpu.VMEM((MR, D), f32),
        pltpu.VMEM((MR,), i32),
        pltpu.SemaphoreType.DMA,
    ],
)
def _mask_kernel(outf_hbm, mask_hbm, y_hbm, om_hbm, ym_hbm,
                 midx, rowb, yb, sem):
    c = lax.axis_index("c")
    s = lax.axis_index("s")
    wid = s * NC + c

    @pl.when(wid < MW)
    def _():
        base = wid * MR
        pltpu.sync_copy(mask_hbm.at[pl.ds(base, MR)], midx)
        pltpu.async_copy(outf_hbm.at[midx], rowb, sem).wait()
        pltpu.sync_copy(rowb, om_hbm.at[pl.ds(base, MR)])
        pltpu.async_copy(y_hbm.at[midx], yb, sem).wait()
        pltpu.sync_copy(yb, ym_hbm.at[pl.ds(base, MR)])


BR = 2000


def _lin1_body(x_ref, w_ref, dp_ref, g_ref, dis_ref):
    deg = dp_ref[0] + dp_ref[1] + 1.0
    dis = jnp.where(deg > 0, lax.rsqrt(jnp.maximum(deg, 1e-12)), 0.0)
    h = jnp.dot(x_ref[...], w_ref[...], preferred_element_type=f32)
    g_ref[...] = h * dis
    dis_ref[...] = dis


def _lin1(x, W1, dp3):
    return pl.pallas_call(
        _lin1_body,
        grid=(N // BR,),
        in_specs=[
            pl.BlockSpec((BR, D), lambda i: (i, 0)),
            pl.BlockSpec((D, D), lambda i: (0, 0)),
            pl.BlockSpec((2, BR, 1), lambda i: (0, i, 0)),
        ],
        out_specs=[
            pl.BlockSpec((BR, D), lambda i: (i, 0)),
            pl.BlockSpec((BR, 1), lambda i: (i, 0)),
        ],
        out_shape=[
            jax.ShapeDtypeStruct((N, D), f32),
            jax.ShapeDtypeStruct((N, 1), f32),
        ],
    )(x, W1, dp3)


def _lin2_body(p_ref, g1_ref, dis_ref, b1_ref, w2_ref, g2_ref):
    t = dis_ref[...] * (p_ref[0] + p_ref[1] + g1_ref[...]) + b1_ref[...]
    z = jnp.maximum(t, 0.0)
    g2_ref[...] = jnp.dot(z, w2_ref[...],
                          preferred_element_type=f32) * dis_ref[...]


def _lin2(p, g1, dis, b1r, W2):
    return pl.pallas_call(
        _lin2_body,
        grid=(N // BR,),
        in_specs=[
            pl.BlockSpec((2, BR, D), lambda i: (0, i, 0)),
            pl.BlockSpec((BR, D), lambda i: (i, 0)),
            pl.BlockSpec((BR, 1), lambda i: (i, 0)),
            pl.BlockSpec((1, D), lambda i: (0, 0)),
            pl.BlockSpec((D, D), lambda i: (0, 0)),
        ],
        out_specs=pl.BlockSpec((BR, D), lambda i: (i, 0)),
        out_shape=jax.ShapeDtypeStruct((N, D), f32),
    )(p, g1, dis, b1r, W2)


def _combine_body(q_ref, g2_ref, dis_ref, b2_ref, o_ref):
    o_ref[...] = dis_ref[...] * (q_ref[0] + q_ref[1] + g2_ref[...]) \
        + b2_ref[...]


def _combine(q, g2, dis, b2r):
    return pl.pallas_call(
        _combine_body,
        grid=(N // BR,),
        in_specs=[
            pl.BlockSpec((2, BR, D), lambda i: (0, i, 0)),
            pl.BlockSpec((BR, D), lambda i: (i, 0)),
            pl.BlockSpec((BR, 1), lambda i: (i, 0)),
            pl.BlockSpec((1, D), lambda i: (0, 0)),
        ],
        out_specs=pl.BlockSpec((BR, D), lambda i: (i, 0)),
        out_shape=jax.ShapeDtypeStruct((N, D), f32),
    )(q, g2, dis, b2r)


def kernel(x, edge_index, edge_weight, mask_idx, y, W1, b1, W2, b2):
    src_idx = edge_index[0]
    dst_idx = edge_index[1]
    dp = _deg_kernel(dst_idx, edge_weight)
    dp3 = dp.reshape(2, N, 1)
    g1, dis = _lin1(x, W1, dp3)
    p = _agg_kernel(src_idx, dst_idx, edge_weight, g1)
    g2 = _lin2(p, g1, dis, b1.reshape(1, D), W2)
    q = _agg_kernel(src_idx, dst_idx, edge_weight, g2)
    outf = _combine(q, g2, dis, b2.reshape(1, D))
    out_m, y_m = _mask_kernel(outf, mask_idx, y)
    return (out_m, y_m)
